# baseline (speedup 1.0000x reference)
import functools

import jax
import jax.numpy as jnp
from jax import lax
from jax.experimental import pallas as pl
from jax.experimental.pallas import tpu as pltpu
from jax.experimental.pallas import tpu_sc as plsc

N = 10000
D = 128
E = 320000
B = 100000
NBIN = 10112
NC = 2
NS = 16
CHUNK = 128
CNT_ROWS = 80

A_CHUNKS = 160
A_GROUP = 16
EW = A_CHUNKS * CHUNK
E_PAD = NS * EW

DEC_CHUNKS = 52
B_PAD = NS * DEC_CHUNKS * CHUNK

_mesh = plsc.VectorSubcoreMesh(core_axis_name="c", subcore_axis_name="s",
                               num_cores=NC, num_subcores=NS)


def _zero_rows(buf, nrows):
    def body(t, _):
        i = t // 8
        k = t % 8
        buf[i, pl.ds(k * 16, 16)] = jnp.zeros((16,), jnp.float32)
        return 0
    lax.fori_loop(0, nrows * 8, body, 0)


def _aggregate(x_hbm, src_hbm, dst_hbm, out_sum,
               acc, src_v, dst_v, buf0, buf1, sem0, sem1, sct0, sct1, sid):
    _zero_rows(buf0, 128)
    base = sid * 632
    for off, n in ((0, 128), (128, 128), (256, 128), (384, 128), (512, 120)):
        pltpu.sync_copy(buf0.at[pl.ds(0, n)], acc.at[pl.ds(base + off, n)])

    plsc.subcore_barrier()

    def group(g, _):
        @pl.when(g > 0)
        def _():
            pltpu.make_async_copy(buf0, acc.at[dst_v.at[0]], sct0).wait()
            pltpu.make_async_copy(buf1, acc.at[dst_v.at[1]], sct1).wait()

        pltpu.sync_copy(src_hbm.at[sid, pl.ds(g * A_GROUP, A_GROUP)], src_v)
        pltpu.sync_copy(dst_hbm.at[sid, pl.ds(g * A_GROUP, A_GROUP)], dst_v)

        def pair(j, _):
            a = 2 * j
            b = a + 1
            first = j == 0

            @pl.when(~first)
            def _():
                pltpu.make_async_copy(buf0, acc.at[dst_v.at[a]], sct0).wait()
            cpa = pltpu.async_copy(x_hbm.at[src_v.at[a]], buf0, sem0)

            @pl.when(~first)
            def _():
                pltpu.make_async_copy(buf1, acc.at[dst_v.at[b]], sct1).wait()
            cpb = pltpu.async_copy(x_hbm.at[src_v.at[b]], buf1, sem1)

            cpa.wait()
            pltpu.async_copy(buf0, acc.at[dst_v.at[a]], sct0, add=True)
            cpb.wait()
            pltpu.async_copy(buf1, acc.at[dst_v.at[b]], sct1, add=True)
            return 0
        lax.fori_loop(0, A_GROUP // 2, pair, 0)
        return 0
    lax.fori_loop(0, A_CHUNKS // A_GROUP, group, 0)

    pltpu.make_async_copy(buf0, acc.at[dst_v.at[0]], sct0).wait()
    pltpu.make_async_copy(buf1, acc.at[dst_v.at[1]], sct1).wait()

    plsc.subcore_barrier()

    pltpu.sync_copy(acc.at[pl.ds(base, 632)], out_sum.at[pl.ds(base, 632)])


@functools.partial(
    pl.kernel,
    out_type=[
        jax.ShapeDtypeStruct((NBIN, D), jnp.float32),
        jax.ShapeDtypeStruct((NBIN, D), jnp.float32),
    ],
    mesh=_mesh,
    scratch_types=[
        pltpu.VMEM_SHARED((NBIN, D), jnp.float32),
        pltpu.VMEM((A_GROUP, CHUNK), jnp.int32),
        pltpu.VMEM((A_GROUP, CHUNK), jnp.int32),
        pltpu.VMEM((CHUNK, D), jnp.float32),
        pltpu.VMEM((CHUNK, D), jnp.float32),
        pltpu.SemaphoreType.DMA,
        pltpu.SemaphoreType.DMA,
        pltpu.SemaphoreType.DMA,
        pltpu.SemaphoreType.DMA,
    ],
)
def _sc_aggregate(x_customer, x_variant, src_vc, dst_vc, src_cv, dst_cv,
                  out_sum_c, out_sum_v,
                  acc, src_v, dst_v, buf0, buf1, sem0, sem1, sct0, sct1):
    cid = lax.axis_index("c")
    sid = lax.axis_index("s")

    @pl.when(cid == 0)
    def _():
        _aggregate(x_variant, src_vc, dst_vc, out_sum_c,
                   acc, src_v, dst_v, buf0, buf1, sem0, sem1, sct0, sct1, sid)

    @pl.when(cid == 1)
    def _():
        _aggregate(x_customer, src_cv, dst_cv, out_sum_v,
                   acc, src_v, dst_v, buf0, buf1, sem0, sem1, sct0, sct1, sid)


def _dec_half(tbl_hbm, idx_hbm, z_hbm, tbl_sp, idx_v, buf0, buf1,
              sem0, sem1, sid):
    @pl.when(sid < 15)
    def _():
        pltpu.sync_copy(tbl_hbm.at[pl.ds(sid * 632, 632)],
                        tbl_sp.at[pl.ds(sid * 632, 632)])

    @pl.when(sid == 15)
    def _():
        pltpu.sync_copy(tbl_hbm.at[pl.ds(15 * 632, N - 15 * 632)],
                        tbl_sp.at[pl.ds(15 * 632, N - 15 * 632)])

    pltpu.sync_copy(idx_hbm.at[sid], idx_v)
    plsc.subcore_barrier()

    base = sid * (DEC_CHUNKS * CHUNK)

    def step(j, _):
        a = 2 * j
        b = a + 1
        cpa = pltpu.async_copy(tbl_sp.at[idx_v.at[a]], buf0, sem0)
        cpb = pltpu.async_copy(tbl_sp.at[idx_v.at[b]], buf1, sem1)
        cpa.wait()
        pltpu.sync_copy(buf0, z_hbm.at[pl.ds(base + a * CHUNK, CHUNK)])
        cpb.wait()
        pltpu.sync_copy(buf1, z_hbm.at[pl.ds(base + b * CHUNK, CHUNK)])
        return 0
    lax.fori_loop(0, DEC_CHUNKS // 2, step, 0)


@functools.partial(
    pl.kernel,
    out_type=[
        jax.ShapeDtypeStruct((B_PAD, D), jnp.float32),
        jax.ShapeDtypeStruct((B_PAD, D), jnp.float32),
    ],
    mesh=_mesh,
    scratch_types=[
        pltpu.VMEM_SHARED((NBIN, D), jnp.float32),
        pltpu.VMEM((DEC_CHUNKS, CHUNK), jnp.int32),
        pltpu.VMEM((CHUNK, D), jnp.float32),
        pltpu.VMEM((CHUNK, D), jnp.float32),
        pltpu.SemaphoreType.DMA,
        pltpu.SemaphoreType.DMA,
    ],
)
def _sc_decoder_gather(hc_hbm, hv_hbm, rows_hbm, cols_hbm, za_hbm, zb_hbm,
                       tbl_sp, idx_v, buf0, buf1, sem0, sem1):
    cid = lax.axis_index("c")
    sid = lax.axis_index("s")

    @pl.when(cid == 0)
    def _():
        _dec_half(hc_hbm, rows_hbm, za_hbm, tbl_sp, idx_v, buf0, buf1,
                  sem0, sem1, sid)

    @pl.when(cid == 1)
    def _():
        _dec_half(hv_hbm, cols_hbm, zb_hbm, tbl_sp, idx_v, buf0, buf1,
                  sem0, sem1, sid)


EB = 4000


def _count_body(dst_vc_ref, dst_cv_ref, cnt_c_ref, cnt_v_ref):
    i = pl.program_id(0)

    @pl.when(i == 0)
    def _():
        cnt_c_ref[...] = jnp.zeros((CNT_ROWS, D), jnp.float32)
        cnt_v_ref[...] = jnp.zeros((CNT_ROWS, D), jnp.float32)

    def one(dst_ref, out_ref):
        d = dst_ref[0, 0, :]
        hi = d >> 7
        lo = d & 127
        oh_hi = (lax.broadcasted_iota(jnp.int32, (CNT_ROWS, EB), 0)
                 == hi[None, :]).astype(jnp.int8)
        oh_lo = (lax.broadcasted_iota(jnp.int32, (EB, D), 1)
                 == lo[:, None]).astype(jnp.int8)
        out_ref[...] += jnp.dot(oh_hi, oh_lo,
                                preferred_element_type=jnp.int32).astype(jnp.float32)

    one(dst_vc_ref, cnt_c_ref)
    one(dst_cv_ref, cnt_v_ref)


def _encode_body(x_c_ref, sum_c_ref, cnt_c_ref, x_v_ref, sum_v_ref, cnt_v_ref,
                 wrc_ref, wnc_ref, wrv_ref, wnv_ref, w0t_ref, w0b_ref, b0_ref,
                 hc_out, hv_out):
    f32 = jnp.float32
    mean_c = sum_c_ref[...] / jnp.maximum(cnt_c_ref[...], 1.0)
    h_c = jnp.maximum(
        jnp.dot(x_c_ref[...], wrc_ref[...], preferred_element_type=f32)
        + jnp.dot(mean_c, wnc_ref[...], preferred_element_type=f32), 0.0)
    hc_out[...] = jnp.dot(h_c, w0t_ref[...], preferred_element_type=f32) + b0_ref[...]
    mean_v = sum_v_ref[...] / jnp.maximum(cnt_v_ref[...], 1.0)
    h_v = jnp.maximum(
        jnp.dot(x_v_ref[...], wrv_ref[...], preferred_element_type=f32)
        + jnp.dot(mean_v, wnv_ref[...], preferred_element_type=f32), 0.0)
    hv_out[...] = jnp.dot(h_v, w0b_ref[...], preferred_element_type=f32)


def _decode_body(za_ref, zb_ref, w1_ref, b1_ref, wo_ref, bo_ref, out_ref):
    f32 = jnp.float32
    z = za_ref[...].astype(f32) + zb_ref[...].astype(f32)
    x1 = jnp.where(z >= 0, z, 0.01 * z)
    t = jnp.dot(x1, w1_ref[...], preferred_element_type=f32) + b1_ref[...]
    x2 = jnp.where(t >= 0, t, 0.01 * t)
    logit = jnp.dot(x2, wo_ref[...], preferred_element_type=f32) + bo_ref[...]
    p = jax.nn.sigmoid(logit)
    out_ref[...] = jnp.concatenate([p, 1.0 - p], axis=1)


def _pad_reshape(a, total, rows, pad_value):
    a = jnp.concatenate([a, jnp.full((total - a.shape[0],), pad_value, a.dtype)])
    return a.reshape(rows, -1, CHUNK)


def kernel(x_customer, x_variant, edge_index_cv, edge_index_vc, edge_label_index,
           W_root_c, W_nb_c, W_root_v, W_nb_v, W0, b0, W1, b1, Wo, bo):
    src_vc = _pad_reshape(edge_index_vc[0], E_PAD, NS, 0)
    dst_vc = _pad_reshape(edge_index_vc[1], E_PAD, NS, N)
    src_cv = _pad_reshape(edge_index_cv[0], E_PAD, NS, 0)
    dst_cv = _pad_reshape(edge_index_cv[1], E_PAD, NS, N)
    rows = _pad_reshape(edge_label_index[0], B_PAD, NS, 0)
    cols = _pad_reshape(edge_label_index[1], B_PAD, NS, 0)

    sum_c, sum_v = _sc_aggregate(
        x_customer, x_variant, src_vc, dst_vc, src_cv, dst_cv)

    cnt_c2, cnt_v2 = pl.pallas_call(
        _count_body,
        grid=(E // EB,),
        in_specs=[
            pl.BlockSpec((1, 1, EB), lambda i: (i, 0, 0)),
            pl.BlockSpec((1, 1, EB), lambda i: (i, 0, 0)),
        ],
        out_specs=[
            pl.BlockSpec((CNT_ROWS, D), lambda i: (0, 0)),
            pl.BlockSpec((CNT_ROWS, D), lambda i: (0, 0)),
        ],
        out_shape=[
            jax.ShapeDtypeStruct((CNT_ROWS, D), jnp.float32),
            jax.ShapeDtypeStruct((CNT_ROWS, D), jnp.float32),
        ],
    )(edge_index_vc[1].reshape(E // EB, 1, EB), edge_index_cv[1].reshape(E // EB, 1, EB))
    cnt_c = cnt_c2.reshape(-1)[:N].reshape(N, 1)
    cnt_v = cnt_v2.reshape(-1)[:N].reshape(N, 1)

    RB = 2000
    hcW0b, hvW0 = pl.pallas_call(
        _encode_body,
        grid=(N // RB,),
        in_specs=[
            pl.BlockSpec((RB, D), lambda i: (i, 0)),
            pl.BlockSpec((RB, D), lambda i: (i, 0)),
            pl.BlockSpec((RB, 1), lambda i: (i, 0)),
            pl.BlockSpec((RB, D), lambda i: (i, 0)),
            pl.BlockSpec((RB, D), lambda i: (i, 0)),
            pl.BlockSpec((RB, 1), lambda i: (i, 0)),
            pl.BlockSpec((D, D), lambda i: (0, 0)),
            pl.BlockSpec((D, D), lambda i: (0, 0)),
            pl.BlockSpec((D, D), lambda i: (0, 0)),
            pl.BlockSpec((D, D), lambda i: (0, 0)),
            pl.BlockSpec((D, D), lambda i: (0, 0)),
            pl.BlockSpec((D, D), lambda i: (0, 0)),
            pl.BlockSpec((1, D), lambda i: (0, 0)),
        ],
        out_specs=[
            pl.BlockSpec((RB, D), lambda i: (i, 0)),
            pl.BlockSpec((RB, D), lambda i: (i, 0)),
        ],
        out_shape=[
            jax.ShapeDtypeStruct((N, D), jnp.float32),
            jax.ShapeDtypeStruct((N, D), jnp.float32),
        ],
    )(x_customer, sum_c[:N], cnt_c, x_variant, sum_v[:N], cnt_v,
      W_root_c, W_nb_c, W_root_v, W_nb_v, W0[:D], W0[D:], b0.reshape(1, D))

    za, zb = _sc_decoder_gather(hcW0b, hvW0, rows, cols)

    DB = 2000
    out = pl.pallas_call(
        _decode_body,
        grid=(B // DB,),
        in_specs=[
            pl.BlockSpec((DB, D), lambda i: (i, 0)),
            pl.BlockSpec((DB, D), lambda i: (i, 0)),
            pl.BlockSpec((D, 64), lambda i: (0, 0)),
            pl.BlockSpec((1, 64), lambda i: (0, 0)),
            pl.BlockSpec((64, 1), lambda i: (0, 0)),
            pl.BlockSpec((1, 1), lambda i: (0, 0)),
        ],
        out_specs=pl.BlockSpec((DB, 2), lambda i: (i, 0)),
        out_shape=jax.ShapeDtypeStruct((B, 2), jnp.float32),
    )(za, zb, W1, b1.reshape(1, 64), Wo, bo.reshape(1, 1))
    return out


# speedup vs baseline: 1.0199x; 1.0199x over previous
import functools

import jax
import jax.numpy as jnp
from jax import lax
from jax.experimental import pallas as pl
from jax.experimental.pallas import tpu as pltpu
from jax.experimental.pallas import tpu_sc as plsc

N = 10000
D = 128
E = 320000
B = 100000
NBIN = 10112
NC = 2
NS = 16
CHUNK = 128
CNT_ROWS = 80

A_CHUNKS = 160
A_GROUP = 16
EW = A_CHUNKS * CHUNK
E_PAD = NS * EW

DEC_CHUNKS = 52
B_PAD = NS * DEC_CHUNKS * CHUNK

_mesh = plsc.VectorSubcoreMesh(core_axis_name="c", subcore_axis_name="s",
                               num_cores=NC, num_subcores=NS)


def _zero_rows(buf, nrows):
    def body(t, _):
        i = t // 8
        k = t % 8
        buf[i, pl.ds(k * 16, 16)] = jnp.zeros((16,), jnp.float32)
        return 0
    lax.fori_loop(0, nrows * 8, body, 0)


def _aggregate(x_hbm, src_hbm, dst_hbm, out_sum,
               acc, src_v, dst_v, buf0, buf1, sem0, sem1, sct0, sct1, sid):
    _zero_rows(buf0, 128)
    base = sid * 632
    for off, n in ((0, 128), (128, 128), (256, 128), (384, 128), (512, 120)):
        pltpu.sync_copy(buf0.at[pl.ds(0, n)], acc.at[pl.ds(base + off, n)])

    plsc.subcore_barrier()

    def group(g, _):
        @pl.when(g > 0)
        def _():
            pltpu.make_async_copy(buf0, acc.at[dst_v.at[0]], sct0).wait()
            pltpu.make_async_copy(buf1, acc.at[dst_v.at[1]], sct1).wait()

        pltpu.sync_copy(src_hbm.at[sid, pl.ds(g * A_GROUP, A_GROUP)], src_v)
        pltpu.sync_copy(dst_hbm.at[sid, pl.ds(g * A_GROUP, A_GROUP)], dst_v)

        def pair(j, _):
            a = 2 * j
            b = a + 1
            first = j == 0

            @pl.when(~first)
            def _():
                pltpu.make_async_copy(buf0, acc.at[dst_v.at[a]], sct0).wait()
            cpa = pltpu.async_copy(x_hbm.at[src_v.at[a]], buf0, sem0)

            @pl.when(~first)
            def _():
                pltpu.make_async_copy(buf1, acc.at[dst_v.at[b]], sct1).wait()
            cpb = pltpu.async_copy(x_hbm.at[src_v.at[b]], buf1, sem1)

            cpa.wait()
            pltpu.async_copy(buf0, acc.at[dst_v.at[a]], sct0, add=True)
            cpb.wait()
            pltpu.async_copy(buf1, acc.at[dst_v.at[b]], sct1, add=True)
            return 0
        lax.fori_loop(0, A_GROUP // 2, pair, 0)
        return 0
    lax.fori_loop(0, A_CHUNKS // A_GROUP, group, 0)

    pltpu.make_async_copy(buf0, acc.at[dst_v.at[0]], sct0).wait()
    pltpu.make_async_copy(buf1, acc.at[dst_v.at[1]], sct1).wait()

    plsc.subcore_barrier()

    pltpu.sync_copy(acc.at[pl.ds(base, 632)], out_sum.at[pl.ds(base, 632)])


@functools.partial(
    pl.kernel,
    out_type=[
        jax.ShapeDtypeStruct((NBIN, D), jnp.float32),
        jax.ShapeDtypeStruct((NBIN, D), jnp.float32),
    ],
    mesh=_mesh,
    scratch_types=[
        pltpu.VMEM_SHARED((NBIN, D), jnp.float32),
        pltpu.VMEM((A_GROUP, CHUNK), jnp.int32),
        pltpu.VMEM((A_GROUP, CHUNK), jnp.int32),
        pltpu.VMEM((CHUNK, D), jnp.float32),
        pltpu.VMEM((CHUNK, D), jnp.float32),
        pltpu.SemaphoreType.DMA,
        pltpu.SemaphoreType.DMA,
        pltpu.SemaphoreType.DMA,
        pltpu.SemaphoreType.DMA,
    ],
)
def _sc_aggregate(x_customer, x_variant, src_vc, dst_vc, src_cv, dst_cv,
                  out_sum_c, out_sum_v,
                  acc, src_v, dst_v, buf0, buf1, sem0, sem1, sct0, sct1):
    cid = lax.axis_index("c")
    sid = lax.axis_index("s")

    @pl.when(cid == 0)
    def _():
        _aggregate(x_variant, src_vc, dst_vc, out_sum_c,
                   acc, src_v, dst_v, buf0, buf1, sem0, sem1, sct0, sct1, sid)

    @pl.when(cid == 1)
    def _():
        _aggregate(x_customer, src_cv, dst_cv, out_sum_v,
                   acc, src_v, dst_v, buf0, buf1, sem0, sem1, sct0, sct1, sid)


def _dec_half(tbl_hbm, idx_hbm, z_hbm, tbl_sp, idx_v, buf0, buf1,
              sem0, sem1, sid):
    @pl.when(sid < 15)
    def _():
        pltpu.sync_copy(tbl_hbm.at[pl.ds(sid * 632, 632)],
                        tbl_sp.at[pl.ds(sid * 632, 632)])

    @pl.when(sid == 15)
    def _():
        pltpu.sync_copy(tbl_hbm.at[pl.ds(15 * 632, N - 15 * 632)],
                        tbl_sp.at[pl.ds(15 * 632, N - 15 * 632)])

    pltpu.sync_copy(idx_hbm.at[sid], idx_v)
    plsc.subcore_barrier()

    base = sid * (DEC_CHUNKS * CHUNK)

    def step(j, _):
        a = 2 * j
        b = a + 1
        cpa = pltpu.async_copy(tbl_sp.at[idx_v.at[a]], buf0, sem0)
        cpb = pltpu.async_copy(tbl_sp.at[idx_v.at[b]], buf1, sem1)
        cpa.wait()
        pltpu.sync_copy(buf0, z_hbm.at[pl.ds(base + a * CHUNK, CHUNK)])
        cpb.wait()
        pltpu.sync_copy(buf1, z_hbm.at[pl.ds(base + b * CHUNK, CHUNK)])
        return 0
    lax.fori_loop(0, DEC_CHUNKS // 2, step, 0)


@functools.partial(
    pl.kernel,
    out_type=[
        jax.ShapeDtypeStruct((B_PAD, D), jnp.float32),
        jax.ShapeDtypeStruct((B_PAD, D), jnp.float32),
    ],
    mesh=_mesh,
    scratch_types=[
        pltpu.VMEM_SHARED((NBIN, D), jnp.float32),
        pltpu.VMEM((DEC_CHUNKS, CHUNK), jnp.int32),
        pltpu.VMEM((CHUNK, D), jnp.float32),
        pltpu.VMEM((CHUNK, D), jnp.float32),
        pltpu.SemaphoreType.DMA,
        pltpu.SemaphoreType.DMA,
    ],
)
def _sc_decoder_gather(hc_hbm, hv_hbm, rows_hbm, cols_hbm, za_hbm, zb_hbm,
                       tbl_sp, idx_v, buf0, buf1, sem0, sem1):
    cid = lax.axis_index("c")
    sid = lax.axis_index("s")

    @pl.when(cid == 0)
    def _():
        _dec_half(hc_hbm, rows_hbm, za_hbm, tbl_sp, idx_v, buf0, buf1,
                  sem0, sem1, sid)

    @pl.when(cid == 1)
    def _():
        _dec_half(hv_hbm, cols_hbm, zb_hbm, tbl_sp, idx_v, buf0, buf1,
                  sem0, sem1, sid)


EB = 4000


def _count_body(dst_vc_ref, dst_cv_ref, cnt_c_ref, cnt_v_ref):
    i = pl.program_id(0)

    @pl.when(i == 0)
    def _():
        cnt_c_ref[...] = jnp.zeros((CNT_ROWS, D), jnp.float32)
        cnt_v_ref[...] = jnp.zeros((CNT_ROWS, D), jnp.float32)

    def one(dst_ref, out_ref):
        d = dst_ref[0, 0, :]
        hi = d >> 7
        lo = d & 127
        oh_hi = (lax.broadcasted_iota(jnp.int32, (CNT_ROWS, EB), 0)
                 == hi[None, :]).astype(jnp.int8)
        oh_lo = (lax.broadcasted_iota(jnp.int32, (EB, D), 1)
                 == lo[:, None]).astype(jnp.int8)
        out_ref[...] += jnp.dot(oh_hi, oh_lo,
                                preferred_element_type=jnp.int32).astype(jnp.float32)

    one(dst_vc_ref, cnt_c_ref)
    one(dst_cv_ref, cnt_v_ref)


def _encode_body(x_c_ref, sum_c_ref, cnt_c_ref, x_v_ref, sum_v_ref, cnt_v_ref,
                 wrc_ref, wnc_ref, wrv_ref, wnv_ref, w0t_ref, w0b_ref, b0_ref,
                 hc_out, hv_out):
    f32 = jnp.float32
    mean_c = sum_c_ref[...] / jnp.maximum(cnt_c_ref[...], 1.0)
    h_c = jnp.maximum(
        jnp.dot(x_c_ref[...], wrc_ref[...], preferred_element_type=f32)
        + jnp.dot(mean_c, wnc_ref[...], preferred_element_type=f32), 0.0)
    hc_out[...] = jnp.dot(h_c, w0t_ref[...], preferred_element_type=f32) + b0_ref[...]
    mean_v = sum_v_ref[...] / jnp.maximum(cnt_v_ref[...], 1.0)
    h_v = jnp.maximum(
        jnp.dot(x_v_ref[...], wrv_ref[...], preferred_element_type=f32)
        + jnp.dot(mean_v, wnv_ref[...], preferred_element_type=f32), 0.0)
    hv_out[...] = jnp.dot(h_v, w0b_ref[...], preferred_element_type=f32)


def _decode_body(za_ref, zb_ref, w1_ref, b1_ref, wo_ref, bo_ref, out_ref):
    f32 = jnp.float32
    z = za_ref[...].astype(f32) + zb_ref[...].astype(f32)
    x1 = jnp.where(z >= 0, z, 0.01 * z)
    t = jnp.dot(x1, w1_ref[...], preferred_element_type=f32) + b1_ref[...]
    x2 = jnp.where(t >= 0, t, 0.01 * t)
    logit = jnp.dot(x2, wo_ref[...], preferred_element_type=f32) + bo_ref[...]
    p = jax.nn.sigmoid(logit)
    out_ref[...] = jnp.concatenate([p, 1.0 - p], axis=1)


def _pad_reshape(a, total, rows, pad_value):
    a = jnp.concatenate([a, jnp.full((total - a.shape[0],), pad_value, a.dtype)])
    return a.reshape(rows, -1, CHUNK)


def kernel(x_customer, x_variant, edge_index_cv, edge_index_vc, edge_label_index,
           W_root_c, W_nb_c, W_root_v, W_nb_v, W0, b0, W1, b1, Wo, bo):
    src_vc = _pad_reshape(edge_index_vc[0], E_PAD, NS, 0)
    dst_vc = _pad_reshape(edge_index_vc[1], E_PAD, NS, N)
    src_cv = _pad_reshape(edge_index_cv[0], E_PAD, NS, 0)
    dst_cv = _pad_reshape(edge_index_cv[1], E_PAD, NS, N)
    rows = _pad_reshape(edge_label_index[0], B_PAD, NS, 0)
    cols = _pad_reshape(edge_label_index[1], B_PAD, NS, 0)

    sum_c, sum_v = _sc_aggregate(
        x_customer, x_variant, src_vc, dst_vc, src_cv, dst_cv)

    cnt_c2, cnt_v2 = pl.pallas_call(
        _count_body,
        grid=(E // EB,),
        in_specs=[
            pl.BlockSpec((1, 1, EB), lambda i: (i, 0, 0)),
            pl.BlockSpec((1, 1, EB), lambda i: (i, 0, 0)),
        ],
        out_specs=[
            pl.BlockSpec((CNT_ROWS, D), lambda i: (0, 0)),
            pl.BlockSpec((CNT_ROWS, D), lambda i: (0, 0)),
        ],
        out_shape=[
            jax.ShapeDtypeStruct((CNT_ROWS, D), jnp.float32),
            jax.ShapeDtypeStruct((CNT_ROWS, D), jnp.float32),
        ],
    )(edge_index_vc[1].reshape(E // EB, 1, EB), edge_index_cv[1].reshape(E // EB, 1, EB))
    cnt_c = cnt_c2.reshape(-1)[:N].reshape(N, 1)
    cnt_v = cnt_v2.reshape(-1)[:N].reshape(N, 1)

    RB = 2000
    hcW0b, hvW0 = pl.pallas_call(
        _encode_body,
        grid=(N // RB,),
        in_specs=[
            pl.BlockSpec((RB, D), lambda i: (i, 0)),
            pl.BlockSpec((RB, D), lambda i: (i, 0)),
            pl.BlockSpec((RB, 1), lambda i: (i, 0)),
            pl.BlockSpec((RB, D), lambda i: (i, 0)),
            pl.BlockSpec((RB, D), lambda i: (i, 0)),
            pl.BlockSpec((RB, 1), lambda i: (i, 0)),
            pl.BlockSpec((D, D), lambda i: (0, 0)),
            pl.BlockSpec((D, D), lambda i: (0, 0)),
            pl.BlockSpec((D, D), lambda i: (0, 0)),
            pl.BlockSpec((D, D), lambda i: (0, 0)),
            pl.BlockSpec((D, D), lambda i: (0, 0)),
            pl.BlockSpec((D, D), lambda i: (0, 0)),
            pl.BlockSpec((1, D), lambda i: (0, 0)),
        ],
        out_specs=[
            pl.BlockSpec((RB, D), lambda i: (i, 0)),
            pl.BlockSpec((RB, D), lambda i: (i, 0)),
        ],
        out_shape=[
            jax.ShapeDtypeStruct((N, D), jnp.float32),
            jax.ShapeDtypeStruct((N, D), jnp.float32),
        ],
    )(x_customer, sum_c[:N], cnt_c, x_variant, sum_v[:N], cnt_v,
      W_root_c, W_nb_c, W_root_v, W_nb_v, W0[:D], W0[D:], b0.reshape(1, D))

    za, zb = _sc_decoder_gather(hcW0b, hvW0, rows, cols)

    DB = 4000
    out = pl.pallas_call(
        _decode_body,
        grid=(B // DB,),
        in_specs=[
            pl.BlockSpec((DB, D), lambda i: (i, 0)),
            pl.BlockSpec((DB, D), lambda i: (i, 0)),
            pl.BlockSpec((D, 64), lambda i: (0, 0)),
            pl.BlockSpec((1, 64), lambda i: (0, 0)),
            pl.BlockSpec((64, 1), lambda i: (0, 0)),
            pl.BlockSpec((1, 1), lambda i: (0, 0)),
        ],
        out_specs=pl.BlockSpec((DB, 2), lambda i: (i, 0)),
        out_shape=jax.ShapeDtypeStruct((B, 2), jnp.float32),
    )(za, zb, W1, b1.reshape(1, 64), Wo, bo.reshape(1, 1))
    return out
